# baseline (device time: 171021 ns/iter reference)
import jax
import jax.numpy as jnp
from jax import lax
from jax.experimental import pallas as pl
from jax.experimental.pallas import tpu as pltpu

N_DEV = 4
N_TOK = 2048
D_IN = 512
D_OUT = 1024
N_EXP = 16
EXP_PER_DEV = N_EXP // N_DEV
CAPACITY = 102


def kernel(x, router_W, route_idx, expert_W):
    del router_W

    my = lax.axis_index("i")

    e_ids = route_idx[:, 0]
    onehot = e_ids[:, None] == jnp.arange(N_EXP)[None, :]
    cum = jnp.cumsum(onehot.astype(jnp.int32), axis=0)
    keep = onehot & (cum <= CAPACITY)
    local_mask = lax.dynamic_slice(
        keep, (0, EXP_PER_DEV * my), (N_TOK, EXP_PER_DEV)
    ).astype(jnp.bfloat16)

    def body(x_ref, mask_ref, w_ref, out_ref, comm_ref, send_sems, recv_sems):
        my_pos = lax.axis_index("i")
        left = (my_pos - 1) % N_DEV
        right = (my_pos + 1) % N_DEV

        barrier_sem = pltpu.get_barrier_semaphore()
        for nbr in [left, right]:
            pl.semaphore_signal(
                barrier_sem, inc=1,
                device_id=(nbr,), device_id_type=pl.DeviceIdType.MESH,
            )
        pl.semaphore_wait(barrier_sem, 2)

        xb = x_ref[...].astype(jnp.bfloat16)
        acc = jnp.zeros((N_TOK, D_OUT), jnp.float32)
        for j in range(EXP_PER_DEV):
            xm = xb * mask_ref[:, j : j + 1]
            wj = w_ref[j].astype(jnp.bfloat16)
            acc = acc + jnp.dot(xm, wj, preferred_element_type=jnp.float32)
        out_ref[...] = acc
        comm_ref[0] = acc.astype(jnp.bfloat16)

        for h in range(N_DEV - 1):
            rdma = pltpu.make_async_remote_copy(
                src_ref=comm_ref.at[h],
                dst_ref=comm_ref.at[h + 1],
                send_sem=send_sems.at[h],
                recv_sem=recv_sems.at[h],
                device_id=(right,),
                device_id_type=pl.DeviceIdType.MESH,
            )
            rdma.start()
            rdma.wait()
            out_ref[...] += comm_ref[h + 1].astype(jnp.float32)

    return pl.pallas_call(
        body,
        out_shape=jax.ShapeDtypeStruct((N_TOK, D_OUT), jnp.float32),
        in_specs=[
            pl.BlockSpec(memory_space=pltpu.VMEM),
            pl.BlockSpec(memory_space=pltpu.VMEM),
            pl.BlockSpec(memory_space=pltpu.VMEM),
        ],
        out_specs=pl.BlockSpec(memory_space=pltpu.VMEM),
        scratch_shapes=[
            pltpu.VMEM((N_DEV, N_TOK, D_OUT), jnp.bfloat16),
            pltpu.SemaphoreType.DMA((N_DEV - 1,)),
            pltpu.SemaphoreType.DMA((N_DEV - 1,)),
        ],
        compiler_params=pltpu.CompilerParams(collective_id=0),
    )(x, local_mask, expert_W)


# device time: 83743 ns/iter; 2.0422x vs baseline; 2.0422x over previous
import jax
import jax.numpy as jnp
from jax import lax
from jax.experimental import pallas as pl
from jax.experimental.pallas import tpu as pltpu

N_DEV = 4
N_TOK = 2048
D_IN = 512
D_OUT = 1024
N_EXP = 16
EXP_PER_DEV = N_EXP // N_DEV
CAPACITY = 102
CAP_PAD = 128
BLOCK_ROWS = EXP_PER_DEV * CAP_PAD


def kernel(x, router_W, route_idx, expert_W):
    del router_W

    my = lax.axis_index("i")

    e = route_idx[:, 0]
    onehot = e[:, None] == jnp.arange(N_EXP)[None, :]
    cum = jnp.cumsum(onehot.astype(jnp.int32), axis=0)
    rank = jnp.take_along_axis(cum, e[:, None], axis=1)[:, 0]
    pos = rank - 1
    kept = rank <= CAPACITY

    dest_block = (e // EXP_PER_DEV).astype(jnp.int32)
    dest_slot = jnp.where(kept, (e % EXP_PER_DEV) * CAP_PAD + pos, -1)

    local_j = e - EXP_PER_DEV * my
    is_mine = (dest_block == my) & kept
    slot_local = jnp.where(is_mine, local_j * CAP_PAD + pos, BLOCK_ROWS)
    tok = (
        jnp.zeros((BLOCK_ROWS,), jnp.int32)
        .at[slot_local]
        .set(jnp.arange(N_TOK, dtype=jnp.int32), mode="drop")
    )
    xc = x[tok]

    dblk = dest_block[:, None]
    dslot = dest_slot[:, None].astype(jnp.int32)

    def body(xc_ref, w_ref, dblk_ref, dslot_ref, out_ref,
             comm_ref, send_sems, recv_sems):
        my_pos = lax.axis_index("i")
        left = (my_pos - 1) % N_DEV
        right = (my_pos + 1) % N_DEV

        barrier_sem = pltpu.get_barrier_semaphore()
        for nbr in [left, right]:
            pl.semaphore_signal(
                barrier_sem, inc=1,
                device_id=(nbr,), device_id_type=pl.DeviceIdType.MESH,
            )
        pl.semaphore_wait(barrier_sem, 2)

        xb = xc_ref[...].astype(jnp.bfloat16)
        for j in range(EXP_PER_DEV):
            cj = jnp.dot(
                xb[j * CAP_PAD : (j + 1) * CAP_PAD],
                w_ref[j].astype(jnp.bfloat16),
                preferred_element_type=jnp.float32,
            )
            comm_ref[0, j * CAP_PAD : (j + 1) * CAP_PAD] = cj.astype(jnp.bfloat16)

        iota = lax.broadcasted_iota(jnp.int32, (N_TOK, BLOCK_ROWS), 1)
        dslot_v = dslot_ref[...]
        dblk_v = dblk_ref[...]

        def scatter_block(block_idx, slot, first):
            p = ((dslot_v == iota) & (dblk_v == block_idx)).astype(jnp.bfloat16)
            contrib = jnp.dot(
                p, comm_ref[slot], preferred_element_type=jnp.float32
            )
            if first:
                out_ref[...] = contrib
            else:
                out_ref[...] += contrib

        scatter_block(my_pos, 0, True)

        for h in range(N_DEV - 1):
            rdma = pltpu.make_async_remote_copy(
                src_ref=comm_ref.at[h],
                dst_ref=comm_ref.at[h + 1],
                send_sem=send_sems.at[h],
                recv_sem=recv_sems.at[h],
                device_id=(right,),
                device_id_type=pl.DeviceIdType.MESH,
            )
            rdma.start()
            rdma.wait()
            scatter_block((my_pos - h - 1) % N_DEV, h + 1, False)

    return pl.pallas_call(
        body,
        out_shape=jax.ShapeDtypeStruct((N_TOK, D_OUT), jnp.float32),
        in_specs=[
            pl.BlockSpec(memory_space=pltpu.VMEM),
            pl.BlockSpec(memory_space=pltpu.VMEM),
            pl.BlockSpec(memory_space=pltpu.VMEM),
            pl.BlockSpec(memory_space=pltpu.VMEM),
        ],
        out_specs=pl.BlockSpec(memory_space=pltpu.VMEM),
        scratch_shapes=[
            pltpu.VMEM((N_DEV, BLOCK_ROWS, D_OUT), jnp.bfloat16),
            pltpu.SemaphoreType.DMA((N_DEV - 1,)),
            pltpu.SemaphoreType.DMA((N_DEV - 1,)),
        ],
        compiler_params=pltpu.CompilerParams(collective_id=0),
    )(xc, expert_W, dblk, dslot)


# device time: 72859 ns/iter; 2.3473x vs baseline; 1.1494x over previous
import jax
import jax.numpy as jnp
from jax import lax
from jax.experimental import pallas as pl
from jax.experimental.pallas import tpu as pltpu

N_DEV = 4
N_TOK = 2048
D_IN = 512
D_OUT = 1024
N_EXP = 16
EXP_PER_DEV = N_EXP // N_DEV
CAPACITY = 102
CAP_PAD = 128
BLOCK_ROWS = EXP_PER_DEV * CAP_PAD


def kernel(x, router_W, route_idx, expert_W):
    del router_W

    my = lax.axis_index("i")

    e = route_idx[:, 0]
    onehot = e[:, None] == jnp.arange(N_EXP)[None, :]
    cum = jnp.cumsum(onehot.astype(jnp.int32), axis=0)
    rank = jnp.sum(cum * onehot, axis=1)
    pos = rank - 1
    kept = rank <= CAPACITY

    dest_block = (e // EXP_PER_DEV).astype(jnp.int32)
    dest_slot = jnp.where(kept, (e % EXP_PER_DEV) * CAP_PAD + pos, -1)

    local_j = e - EXP_PER_DEV * my
    is_mine = (dest_block == my) & kept
    slot_local = jnp.where(is_mine, local_j * CAP_PAD + pos, -1)

    dblk = dest_block[:, None]
    dslot = dest_slot[:, None].astype(jnp.int32)
    slot_row = slot_local[None, :].astype(jnp.int32)

    def body(x_ref, w_ref, dblk_ref, dslot_ref, slot_row_ref, out_ref,
             comm_ref, send_sems, recv_sems):
        my_pos = lax.axis_index("i")
        left = (my_pos - 1) % N_DEV
        right = (my_pos + 1) % N_DEV

        barrier_sem = pltpu.get_barrier_semaphore()
        for nbr in [left, right]:
            pl.semaphore_signal(
                barrier_sem, inc=1,
                device_id=(nbr,), device_id_type=pl.DeviceIdType.MESH,
            )
        pl.semaphore_wait(barrier_sem, 2)

        gather_iota = lax.broadcasted_iota(jnp.int32, (BLOCK_ROWS, N_TOK), 0)
        g = (slot_row_ref[...] == gather_iota).astype(jnp.bfloat16)
        xb = jnp.dot(
            g, x_ref[...].astype(jnp.bfloat16),
            preferred_element_type=jnp.float32,
        ).astype(jnp.bfloat16)

        for j in range(EXP_PER_DEV):
            cj = jnp.dot(
                xb[j * CAP_PAD : (j + 1) * CAP_PAD],
                w_ref[j].astype(jnp.bfloat16),
                preferred_element_type=jnp.float32,
            )
            comm_ref[0, j * CAP_PAD : (j + 1) * CAP_PAD] = cj.astype(jnp.bfloat16)

        iota = lax.broadcasted_iota(jnp.int32, (N_TOK, BLOCK_ROWS), 1)
        dslot_v = dslot_ref[...]
        dblk_v = dblk_ref[...]

        def scatter_block(block_idx, slot, first):
            p = ((dslot_v == iota) & (dblk_v == block_idx)).astype(jnp.bfloat16)
            contrib = jnp.dot(
                p, comm_ref[slot], preferred_element_type=jnp.float32
            )
            if first:
                out_ref[...] = contrib
            else:
                out_ref[...] += contrib

        scatter_block(my_pos, 0, True)

        for h in range(N_DEV - 1):
            rdma = pltpu.make_async_remote_copy(
                src_ref=comm_ref.at[h],
                dst_ref=comm_ref.at[h + 1],
                send_sem=send_sems.at[h],
                recv_sem=recv_sems.at[h],
                device_id=(right,),
                device_id_type=pl.DeviceIdType.MESH,
            )
            rdma.start()
            rdma.wait()
            scatter_block((my_pos - h - 1) % N_DEV, h + 1, False)

    return pl.pallas_call(
        body,
        out_shape=jax.ShapeDtypeStruct((N_TOK, D_OUT), jnp.float32),
        in_specs=[pl.BlockSpec(memory_space=pltpu.VMEM)] * 5,
        out_specs=pl.BlockSpec(memory_space=pltpu.VMEM),
        scratch_shapes=[
            pltpu.VMEM((N_DEV, BLOCK_ROWS, D_OUT), jnp.bfloat16),
            pltpu.SemaphoreType.DMA((N_DEV - 1,)),
            pltpu.SemaphoreType.DMA((N_DEV - 1,)),
        ],
        compiler_params=pltpu.CompilerParams(collective_id=0),
    )(x, expert_W, dblk, dslot, slot_row)


# device time: 47005 ns/iter; 3.6384x vs baseline; 1.5500x over previous
import jax
import jax.numpy as jnp
from jax import lax
from jax.experimental import pallas as pl
from jax.experimental.pallas import tpu as pltpu

N_DEV = 4
N_TOK = 2048
D_IN = 512
D_OUT = 1024
N_EXP = 16
EXP_PER_DEV = N_EXP // N_DEV
CAPACITY = 102
CAP_PAD = 104
BLOCK_ROWS = EXP_PER_DEV * CAP_PAD


def kernel(x, router_W, route_idx, expert_W):
    del router_W

    my = lax.axis_index("i")

    e = route_idx[:, 0]
    onehot = e[:, None] == jnp.arange(N_EXP)[None, :]
    cum = jnp.cumsum(onehot.astype(jnp.int32), axis=0)
    rank = jnp.sum(cum * onehot, axis=1)
    pos = rank - 1
    kept = rank <= CAPACITY

    dest_block = (e // EXP_PER_DEV).astype(jnp.int32)
    dest_slot = jnp.where(kept, (e % EXP_PER_DEV) * CAP_PAD + pos, -1)

    local_j = e - EXP_PER_DEV * my
    is_mine = (dest_block == my) & kept
    slot_local = jnp.where(is_mine, local_j * CAP_PAD + pos, -1)

    dblk = dest_block[:, None]
    dslot = dest_slot[:, None].astype(jnp.int32)
    slot_row = slot_local[None, :].astype(jnp.int32)

    def body(x_ref, w_ref, dblk_ref, dslot_ref, slot_row_ref, out_ref,
             own_ref, comm_ref, send_sems, recv_sems):
        my_pos = lax.axis_index("i")
        left = (my_pos - 1) % N_DEV
        right = (my_pos + 1) % N_DEV
        diag = (my_pos + 2) % N_DEV

        barrier_sem = pltpu.get_barrier_semaphore()
        for nbr in [left, right, diag]:
            pl.semaphore_signal(
                barrier_sem, inc=1,
                device_id=(nbr,), device_id_type=pl.DeviceIdType.MESH,
            )
        pl.semaphore_wait(barrier_sem, 3)

        gather_iota = lax.broadcasted_iota(jnp.int32, (BLOCK_ROWS, N_TOK), 0)
        g = (slot_row_ref[...] == gather_iota).astype(jnp.bfloat16)
        xb = jnp.dot(
            g, x_ref[...].astype(jnp.bfloat16),
            preferred_element_type=jnp.float32,
        ).astype(jnp.bfloat16)

        for j in range(EXP_PER_DEV):
            cj = jnp.dot(
                xb[j * CAP_PAD : (j + 1) * CAP_PAD],
                w_ref[j].astype(jnp.bfloat16),
                preferred_element_type=jnp.float32,
            )
            own_ref[j * CAP_PAD : (j + 1) * CAP_PAD, :] = cj.astype(jnp.bfloat16)

        rdmas = []
        for peer, slot in ((right, 0), (left, 1), (diag, 2)):
            rdma = pltpu.make_async_remote_copy(
                src_ref=own_ref,
                dst_ref=comm_ref.at[slot],
                send_sem=send_sems.at[slot],
                recv_sem=recv_sems.at[slot],
                device_id=(peer,),
                device_id_type=pl.DeviceIdType.MESH,
            )
            rdma.start()
            rdmas.append(rdma)

        scat_iota = lax.broadcasted_iota(jnp.int32, (N_TOK, BLOCK_ROWS), 1)
        dslot_v = dslot_ref[...]
        dblk_v = dblk_ref[...]

        def scatter_block(origin, block_ref, first):
            p = ((dslot_v == scat_iota) & (dblk_v == origin)).astype(jnp.bfloat16)
            contrib = jnp.dot(p, block_ref[...], preferred_element_type=jnp.float32)
            if first:
                out_ref[...] = contrib
            else:
                out_ref[...] += contrib

        scatter_block(my_pos, own_ref, True)
        for origin, slot in ((left, 0), (right, 1), (diag, 2)):
            rdmas[slot].wait_recv()
            scatter_block(origin, comm_ref.at[slot], False)

        for rdma in rdmas:
            rdma.wait_send()

    return pl.pallas_call(
        body,
        out_shape=jax.ShapeDtypeStruct((N_TOK, D_OUT), jnp.float32),
        in_specs=[pl.BlockSpec(memory_space=pltpu.VMEM)] * 5,
        out_specs=pl.BlockSpec(memory_space=pltpu.VMEM),
        scratch_shapes=[
            pltpu.VMEM((BLOCK_ROWS, D_OUT), jnp.bfloat16),
            pltpu.VMEM((N_DEV - 1, BLOCK_ROWS, D_OUT), jnp.bfloat16),
            pltpu.SemaphoreType.DMA((N_DEV - 1,)),
            pltpu.SemaphoreType.DMA((N_DEV - 1,)),
        ],
        compiler_params=pltpu.CompilerParams(collective_id=0),
    )(x, expert_W, dblk, dslot, slot_row)
